# baseline (speedup 1.0000x reference)
import functools

import jax
import jax.numpy as jnp
from jax import lax
from jax.experimental import pallas as pl
from jax.experimental.pallas import tpu as pltpu
from jax.experimental.pallas import tpu_sc as plsc

_COORDS_RANGE = 10.0
_NC = 2
_NS = 16
_NW = _NC * _NS
_C = 128


def _silu(v):
    return v * jax.nn.sigmoid(v)


def _mesh():
    return plsc.VectorSubcoreMesh(core_axis_name="c", subcore_axis_name="s",
                                  num_cores=_NC, num_subcores=_NS)


_SC_TILED = pltpu.CompilerParams(use_tc_tiling_on_sc=True)
_SC_LINEAR = pltpu.CompilerParams(use_tc_tiling_on_sc=False)


def _stage_rows(src_hbm, spm, sid, N):
    ch = (N // _NS) & ~7
    left = N - _NS * ch
    pltpu.sync_copy(src_hbm.at[pl.ds(sid * ch, ch)],
                    spm.at[pl.ds(sid * ch, ch)])
    if left:
        @pl.when(sid == _NS - 1)
        def _():
            pltpu.sync_copy(src_hbm.at[pl.ds(_NS * ch, left)],
                            spm.at[pl.ds(_NS * ch, left)])


def _gather_sym_body(K2, N, eA, eB, idxs2, idxd2, gA_out, gB_out,
                     iv, spm, buf, sem0, sem1, semw0, semw1):
    cid = lax.axis_index("c")
    sid = lax.axis_index("s")
    sems = (sem0, sem1)
    semw = (semw0, semw1)

    def run(tab, idx2, out):
        _stage_rows(tab, spm, sid, N)
        plsc.subcore_barrier()
        base = sid * (K2 * _C)

        def issue(j, b):
            pltpu.sync_copy(idx2.at[sid, j], iv.at[b])
            pltpu.async_copy(spm.at[iv.at[b]], buf.at[b], sems[b])

        def drain(b):
            pltpu.make_async_copy(spm.at[pl.ds(0, _C)], buf.at[b],
                                  sems[b]).wait()

        def drain_wb(b):
            pltpu.make_async_copy(buf.at[b], out.at[pl.ds(0, _C)],
                                  semw[b]).wait()

        issue(0, 0)

        def step(i2, _):
            for b in (0, 1):
                j = i2 * 2 + b

                @pl.when((j + 1 < K2) & (j >= 1))
                def _():
                    drain_wb(1 - b)

                @pl.when(j + 1 < K2)
                def _():
                    issue(j + 1, 1 - b)

                drain(b)
                pltpu.async_copy(buf.at[b], out.at[pl.ds(base + j * _C, _C)],
                                 semw[b])
            return _

        lax.fori_loop(0, K2 // 2, step, None)
        drain_wb(0)
        drain_wb(1)

    @pl.when(cid == 0)
    def _():
        run(eA, idxs2, gA_out)

    @pl.when(cid == 1)
    def _():
        run(eB, idxd2, gB_out)


def _sc_gather(eA, eB, idxs2, idxd2, K2, Ep):
    H = eA.shape[1]
    N = eA.shape[0]
    kfn = pl.kernel(
        functools.partial(_gather_sym_body, K2, N),
        out_type=[jax.ShapeDtypeStruct((Ep, H), jnp.float32),
                  jax.ShapeDtypeStruct((Ep, H), jnp.float32)],
        mesh=_mesh(),
        compiler_params=_SC_TILED,
        scratch_types=[
            pltpu.VMEM((2, _C), jnp.int32),
            pltpu.VMEM_SHARED((N, H), jnp.float32),
            pltpu.VMEM((2, _C, H), jnp.float32),
            pltpu.SemaphoreType.DMA, pltpu.SemaphoreType.DMA,
            pltpu.SemaphoreType.DMA, pltpu.SemaphoreType.DMA,
        ],
    )
    return kfn(eA, eB, idxs2, idxd2)


def _gatherx_body(K, N, XP, idxs3, idxd3, gx_out,
                  ivs, ivd, spm, bufS, bufD, sem0, sem1, semw0, semw1):
    cid = lax.axis_index("c")
    sid = lax.axis_index("s")
    wid = cid * _NS + sid
    base = wid * (K * _C)
    _stage_rows(XP, spm, sid, N)
    plsc.subcore_barrier()
    pltpu.sync_copy(idxs3.at[wid], ivs)
    pltpu.sync_copy(idxd3.at[wid], ivd)
    sems = (sem0, sem1)
    semw = (semw0, semw1)

    def issue(j, b):
        pltpu.async_copy(spm.at[ivs.at[j]], bufS.at[b], sems[b])
        pltpu.async_copy(spm.at[ivd.at[j]], bufD.at[b], sems[b])

    def drain(b):
        pltpu.make_async_copy(spm.at[pl.ds(0, _C)], bufS.at[b],
                              sems[b]).wait()
        pltpu.make_async_copy(spm.at[pl.ds(0, _C)], bufD.at[b],
                              sems[b]).wait()

    def drain_wb(b):
        pltpu.make_async_copy(bufS.at[b], gx_out.at[pl.ds(0, _C)],
                              semw[b]).wait()

    issue(0, 0)

    def step(i2, _):
        for b in (0, 1):
            j = i2 * 2 + b

            @pl.when((j + 1 < K) & (j >= 1))
            def _():
                drain_wb(1 - b)

            @pl.when(j + 1 < K)
            def _():
                issue(j + 1, 1 - b)

            drain(b)

            def row(r4, carry):
                for rr in range(4):
                    r = r4 * 4 + rr
                    bufS[b, r, :] = bufS[b, r, :] - bufD[b, r, :]
                return carry

            lax.fori_loop(0, _C // 4, row, None)
            pltpu.async_copy(bufS.at[b], gx_out.at[pl.ds(base + j * _C, _C)],
                             semw[b])
        return _

    lax.fori_loop(0, K // 2, step, None)
    drain_wb(0)
    drain_wb(1)


def _sc_gatherx(XP, idxs3, idxd3, K, Ep):
    N = XP.shape[0]
    kfn = pl.kernel(
        functools.partial(_gatherx_body, K, N),
        out_type=jax.ShapeDtypeStruct((Ep, 16), jnp.float32),
        mesh=_mesh(),
        compiler_params=_SC_LINEAR,
        scratch_types=[
            pltpu.VMEM((K, _C), jnp.int32),
            pltpu.VMEM((K, _C), jnp.int32),
            pltpu.VMEM_SHARED((N, 16), jnp.float32),
            pltpu.VMEM((2, _C, 16), jnp.float32),
            pltpu.VMEM((2, _C, 16), jnp.float32),
            pltpu.SemaphoreType.DMA, pltpu.SemaphoreType.DMA,
            pltpu.SemaphoreType.DMA, pltpu.SemaphoreType.DMA,
        ],
    )
    return kfn(XP, idxs3, idxd3)


def _scatter_body(K, Np, D, payload, idxd3, zz, acc_out,
                  ivd, buf, acc, sem0, sem1):
    cid = lax.axis_index("c")
    sid = lax.axis_index("s")
    wid = cid * _NS + sid
    base = wid * (K * _C)
    rows = Np // _NS
    zbase = sid * rows
    nfull = rows // _C
    for t in range(nfull):
        pltpu.sync_copy(zz.at[pl.ds(0, _C)], acc.at[pl.ds(zbase + t * _C, _C)])
    rem = rows - nfull * _C
    if rem:
        pltpu.sync_copy(zz.at[pl.ds(0, rem)],
                        acc.at[pl.ds(zbase + nfull * _C, rem)])
    plsc.subcore_barrier()

    pltpu.sync_copy(idxd3.at[wid], ivd)
    sems = (sem0, sem1)

    def issue(j, b):
        pltpu.async_copy(payload.at[pl.ds(base + j * _C, _C)], buf.at[b],
                         sems[b])

    def drain(b):
        pltpu.make_async_copy(payload.at[pl.ds(0, _C)], buf.at[b],
                              sems[b]).wait()

    issue(0, 0)

    def step(i2, _):
        for b in (0, 1):
            j = i2 * 2 + b

            @pl.when(j + 1 < K)
            def _():
                issue(j + 1, 1 - b)

            drain(b)
            pltpu.sync_copy(buf.at[b], acc.at[ivd.at[j]], add=True)
        return _

    lax.fori_loop(0, K // 2, step, None)
    plsc.subcore_barrier()
    pltpu.sync_copy(acc.at[pl.ds(zbase, rows)],
                    acc_out.at[cid, pl.ds(zbase, rows)])


def _sc_scatter(payload, idxd3, zz, K, Np, tiled):
    D = payload.shape[1]
    kfn = pl.kernel(
        functools.partial(_scatter_body, K, Np, D),
        out_type=jax.ShapeDtypeStruct((_NC, Np, D), jnp.float32),
        mesh=_mesh(),
        compiler_params=_SC_TILED if tiled else _SC_LINEAR,
        scratch_types=[
            pltpu.VMEM((K, _C), jnp.int32),
            pltpu.VMEM((2, _C, D), jnp.float32),
            pltpu.VMEM_SHARED((Np, D), jnp.float32),
            pltpu.SemaphoreType.DMA, pltpu.SemaphoreType.DMA,
        ],
    )
    return kfn(payload, idxd3, zz)


def _edge_body(gA, gB, gx, wr, b1, W2T, b2, cW1T, cb1, cW2r,
               msgh_out, small_out):
    xdiff = gx[:, 0:3]
    radial = jnp.sum(xdiff * xdiff, axis=1, keepdims=True)
    u = xdiff / (jnp.sqrt(radial) + 1e-30)
    pre = gA[:] + gB[:] + radial * wr[:] + b1[:]
    m = _silu(pre)
    mh = _silu(jnp.dot(m, W2T[:], preferred_element_type=jnp.float32) + b2[:])
    c = _silu(jnp.dot(mh, cW1T[:], preferred_element_type=jnp.float32) + cb1[:])
    t = jnp.tanh(jnp.sum(c * cW2r[:], axis=1, keepdims=True))
    msgh_out[:] = mh
    mx = t * u * _COORDS_RANGE
    ones = jnp.ones_like(t)
    small_out[:] = jnp.concatenate(
        [mx, ones, jnp.zeros((t.shape[0], 4), t.dtype)], axis=1)


def _edge_mlp(gA, gB, gx, p):
    Ep, H = gA.shape
    wr = p['eW1'][:, -1].reshape(1, H)
    b1 = p['eb1'].reshape(1, H)
    blk = 2560
    full = lambda shape: pl.BlockSpec(shape, lambda i: (0, 0))
    return pl.pallas_call(
        _edge_body,
        grid=(Ep // blk,),
        in_specs=[
            pl.BlockSpec((blk, H), lambda i: (i, 0)),
            pl.BlockSpec((blk, H), lambda i: (i, 0)),
            pl.BlockSpec((blk, 16), lambda i: (i, 0)),
            full((1, H)), full((1, H)), full((H, H)), full((1, H)),
            full((H, H)), full((1, H)), full((1, H)),
        ],
        out_specs=[
            pl.BlockSpec((blk, H), lambda i: (i, 0)),
            pl.BlockSpec((blk, 8), lambda i: (i, 0)),
        ],
        out_shape=[
            jax.ShapeDtypeStruct((Ep, H), jnp.float32),
            jax.ShapeDtypeStruct((Ep, 8), jnp.float32),
        ],
    )(gA, gB, gx, wr, b1, p['eW2'].T, p['eb2'].reshape(1, H),
      p['cW1'].T, p['cb1'].reshape(1, H), p['cW2'].reshape(1, H))


def _proj_body(h, WaT, WbT, eA_out, eB_out):
    eA_out[:] = jnp.dot(h[:], WaT[:], preferred_element_type=jnp.float32)
    eB_out[:] = jnp.dot(h[:], WbT[:], preferred_element_type=jnp.float32)


def _edge_proj(h, eW1):
    N, F = h.shape
    H = eW1.shape[0]
    WaT = eW1[:, :F].T
    WbT = eW1[:, F:2 * F].T
    Fp = max(8, -(-F // 8) * 8)
    if Fp != F:
        h = jnp.pad(h, ((0, 0), (0, Fp - F)))
        WaT = jnp.pad(WaT, ((0, Fp - F), (0, 0)))
        WbT = jnp.pad(WbT, ((0, Fp - F), (0, 0)))
    blk = 2000 if N % 2000 == 0 else N
    return pl.pallas_call(
        _proj_body,
        grid=(N // blk,),
        in_specs=[
            pl.BlockSpec((blk, Fp), lambda i: (i, 0)),
            pl.BlockSpec((Fp, H), lambda i: (0, 0)),
            pl.BlockSpec((Fp, H), lambda i: (0, 0)),
        ],
        out_specs=[
            pl.BlockSpec((blk, H), lambda i: (i, 0)),
            pl.BlockSpec((blk, H), lambda i: (i, 0)),
        ],
        out_shape=[
            jax.ShapeDtypeStruct((N, H), jnp.float32),
            jax.ShapeDtypeStruct((N, H), jnp.float32),
        ],
    )(h, WaT, WbT)


def _node_body(h, hacc, W1aT, W1bT, b1, W2T, b2, hout):
    hn = hacc[0] + hacc[1]
    z = (jnp.dot(h[:], W1aT[:], preferred_element_type=jnp.float32)
         + jnp.dot(hn, W1bT[:], preferred_element_type=jnp.float32) + b1[:])
    z = _silu(z)
    hout[:] = jnp.dot(z, W2T[:], preferred_element_type=jnp.float32) + b2[:]


def _node_update(h, hacc, p):
    N, F = h.shape
    H = hacc.shape[2]
    OUTF = p['nW2'].shape[0]
    W1aT = p['nW1'][:, :F].T
    W1bT = p['nW1'][:, F:].T
    b1 = p['nb1'].reshape(1, -1)
    W2T = p['nW2'].T
    b2 = p['nb2'].reshape(1, -1)
    Fp = max(8, -(-F // 8) * 8)
    if Fp != F:
        h = jnp.pad(h, ((0, 0), (0, Fp - F)))
        W1aT = jnp.pad(W1aT, ((0, Fp - F), (0, 0)))
    blk = 2000 if N % 2000 == 0 else N
    return pl.pallas_call(
        _node_body,
        grid=(N // blk,),
        in_specs=[
            pl.BlockSpec((blk, Fp), lambda i: (i, 0)),
            pl.BlockSpec((2, blk, H), lambda i: (0, i, 0)),
            pl.BlockSpec((Fp, H), lambda i: (0, 0)),
            pl.BlockSpec((H, H), lambda i: (0, 0)),
            pl.BlockSpec((1, H), lambda i: (0, 0)),
            pl.BlockSpec((H, OUTF), lambda i: (0, 0)),
            pl.BlockSpec((1, OUTF), lambda i: (0, 0)),
        ],
        out_specs=pl.BlockSpec((blk, OUTF), lambda i: (i, 0)),
        out_shape=jax.ShapeDtypeStruct((N, OUTF), jnp.float32),
    )(h, hacc, W1aT, W1bT, b1, W2T, b2)


def kernel(node_feat, coord_feat, edge_index, params):
    src = edge_index[0].astype(jnp.int32)
    dst = edge_index[1].astype(jnp.int32)
    N = node_feat.shape[0]
    E = src.shape[0]

    K = -(-E // (_NW * _C))
    K = -(-K // 10) * 10
    Ep = _NW * _C * K
    K2 = 2 * K
    Np = -(-(N + 1) // _C) * _C

    pad = Ep - E
    srcp = jnp.pad(src, (0, pad))
    dstp = jnp.pad(dst, (0, pad))
    srcg2 = srcp.reshape(_NS, K2, _C)
    dstg2 = dstp.reshape(_NS, K2, _C)
    srcg3 = srcp.reshape(_NW, K, _C)
    dstg3 = dstp.reshape(_NW, K, _C)
    dsts3 = jnp.pad(dst, (0, pad), constant_values=N).reshape(_NW, K, _C)
    zh = jnp.zeros((_C, 128), jnp.float32)
    zs = jnp.zeros((_C, 8), jnp.float32)

    h, x = node_feat, coord_feat
    for p in params:
        eA, eB = _edge_proj(h, p['eW1'])
        XP = jnp.pad(x, ((0, 0), (0, 13)))
        gA, gB = _sc_gather(eA, eB, srcg2, dstg2, K2, Ep)
        gx = _sc_gatherx(XP, srcg3, dstg3, K, Ep)
        msgh, small = _edge_mlp(gA, gB, gx, p)
        hacc = _sc_scatter(msgh, dsts3, zh, K, Np, tiled=True)
        small, hacc = lax.optimization_barrier((small, hacc))
        sacc = _sc_scatter(small, dsts3, zs, K, Np, tiled=False)
        sm = sacc[0, :N] + sacc[1, :N]
        deg = jnp.maximum(sm[:, 3:4], 1.0)
        x = x + sm[:, 0:3] / deg
        h = _node_update(h, hacc, p)
    return (h, x)


# speedup vs baseline: 1.0147x; 1.0147x over previous
import functools

import jax
import jax.numpy as jnp
from jax import lax
from jax.experimental import pallas as pl
from jax.experimental.pallas import tpu as pltpu
from jax.experimental.pallas import tpu_sc as plsc

_COORDS_RANGE = 10.0
_NC = 2
_NS = 16
_NW = _NC * _NS
_C = 128


def _silu(v):
    return v * jax.nn.sigmoid(v)


def _mesh():
    return plsc.VectorSubcoreMesh(core_axis_name="c", subcore_axis_name="s",
                                  num_cores=_NC, num_subcores=_NS)


_SC_TILED = pltpu.CompilerParams(use_tc_tiling_on_sc=True)
_SC_LINEAR = pltpu.CompilerParams(use_tc_tiling_on_sc=False)


def _stage_rows(src_hbm, spm, sid, N):
    ch = (N // _NS) & ~7
    left = N - _NS * ch
    pltpu.sync_copy(src_hbm.at[pl.ds(sid * ch, ch)],
                    spm.at[pl.ds(sid * ch, ch)])
    if left:
        @pl.when(sid == _NS - 1)
        def _():
            pltpu.sync_copy(src_hbm.at[pl.ds(_NS * ch, left)],
                            spm.at[pl.ds(_NS * ch, left)])


def _gather_sym_body(K2, N, eA, eB, idxs2, idxd2, gA_out, gB_out,
                     iv, spm, buf, sem0, sem1, semw0, semw1):
    cid = lax.axis_index("c")
    sid = lax.axis_index("s")
    sems = (sem0, sem1)
    semw = (semw0, semw1)

    def run(tab, idx2, out):
        _stage_rows(tab, spm, sid, N)
        plsc.subcore_barrier()
        base = sid * (K2 * _C)

        def issue(j, b):
            pltpu.sync_copy(idx2.at[sid, j], iv.at[b])
            pltpu.async_copy(spm.at[iv.at[b]], buf.at[b], sems[b])

        def drain(b):
            pltpu.make_async_copy(spm.at[pl.ds(0, _C)], buf.at[b],
                                  sems[b]).wait()

        def drain_wb(b):
            pltpu.make_async_copy(buf.at[b], out.at[pl.ds(0, _C)],
                                  semw[b]).wait()

        issue(0, 0)

        def step(i2, _):
            for b in (0, 1):
                j = i2 * 2 + b

                @pl.when((j + 1 < K2) & (j >= 1))
                def _():
                    drain_wb(1 - b)

                @pl.when(j + 1 < K2)
                def _():
                    issue(j + 1, 1 - b)

                drain(b)
                pltpu.async_copy(buf.at[b], out.at[pl.ds(base + j * _C, _C)],
                                 semw[b])
            return _

        lax.fori_loop(0, K2 // 2, step, None)
        drain_wb(0)
        drain_wb(1)

    @pl.when(cid == 0)
    def _():
        run(eA, idxs2, gA_out)

    @pl.when(cid == 1)
    def _():
        run(eB, idxd2, gB_out)


def _sc_gather(eA, eB, idxs2, idxd2, K2, Ep):
    H = eA.shape[1]
    N = eA.shape[0]
    kfn = pl.kernel(
        functools.partial(_gather_sym_body, K2, N),
        out_type=[jax.ShapeDtypeStruct((Ep, H), jnp.float32),
                  jax.ShapeDtypeStruct((Ep, H), jnp.float32)],
        mesh=_mesh(),
        compiler_params=_SC_TILED,
        scratch_types=[
            pltpu.VMEM((2, _C), jnp.int32),
            pltpu.VMEM_SHARED((N, H), jnp.float32),
            pltpu.VMEM((2, _C, H), jnp.float32),
            pltpu.SemaphoreType.DMA, pltpu.SemaphoreType.DMA,
            pltpu.SemaphoreType.DMA, pltpu.SemaphoreType.DMA,
        ],
    )
    return kfn(eA, eB, idxs2, idxd2)


def _gatherx_body(K, N, XP, idxs3, idxd3, gx_out,
                  ivs, ivd, spm, bufS, bufD, sem0, sem1, semw0, semw1):
    cid = lax.axis_index("c")
    sid = lax.axis_index("s")
    wid = cid * _NS + sid
    base = wid * (K * _C)
    _stage_rows(XP, spm, sid, N)
    plsc.subcore_barrier()
    pltpu.sync_copy(idxs3.at[wid], ivs)
    pltpu.sync_copy(idxd3.at[wid], ivd)
    sems = (sem0, sem1)
    semw = (semw0, semw1)

    def issue(j, b):
        pltpu.async_copy(spm.at[ivs.at[j]], bufS.at[b], sems[b])
        pltpu.async_copy(spm.at[ivd.at[j]], bufD.at[b], sems[b])

    def drain(b):
        pltpu.make_async_copy(spm.at[pl.ds(0, _C)], bufS.at[b],
                              sems[b]).wait()
        pltpu.make_async_copy(spm.at[pl.ds(0, _C)], bufD.at[b],
                              sems[b]).wait()

    def drain_wb(b):
        pltpu.make_async_copy(bufS.at[b], gx_out.at[pl.ds(0, _C)],
                              semw[b]).wait()

    issue(0, 0)

    def step(i2, _):
        for b in (0, 1):
            j = i2 * 2 + b

            @pl.when((j + 1 < K) & (j >= 1))
            def _():
                drain_wb(1 - b)

            @pl.when(j + 1 < K)
            def _():
                issue(j + 1, 1 - b)

            drain(b)

            def row(r4, carry):
                for rr in range(4):
                    r = r4 * 4 + rr
                    bufS[b, r, :] = bufS[b, r, :] - bufD[b, r, :]
                return carry

            lax.fori_loop(0, _C // 4, row, None)
            pltpu.async_copy(bufS.at[b], gx_out.at[pl.ds(base + j * _C, _C)],
                             semw[b])
        return _

    lax.fori_loop(0, K // 2, step, None)
    drain_wb(0)
    drain_wb(1)


def _sc_gatherx(XP, idxs3, idxd3, K, Ep):
    N = XP.shape[0]
    kfn = pl.kernel(
        functools.partial(_gatherx_body, K, N),
        out_type=jax.ShapeDtypeStruct((Ep, 16), jnp.float32),
        mesh=_mesh(),
        compiler_params=_SC_LINEAR,
        scratch_types=[
            pltpu.VMEM((K, _C), jnp.int32),
            pltpu.VMEM((K, _C), jnp.int32),
            pltpu.VMEM_SHARED((N, 16), jnp.float32),
            pltpu.VMEM((2, _C, 16), jnp.float32),
            pltpu.VMEM((2, _C, 16), jnp.float32),
            pltpu.SemaphoreType.DMA, pltpu.SemaphoreType.DMA,
            pltpu.SemaphoreType.DMA, pltpu.SemaphoreType.DMA,
        ],
    )
    return kfn(XP, idxs3, idxd3)


def _scatter_body(K, Np, D, payload, idxd3, zz, acc_out,
                  ivd, buf, acc, sem0, sem1):
    cid = lax.axis_index("c")
    sid = lax.axis_index("s")
    wid = cid * _NS + sid
    base = wid * (K * _C)
    rows = Np // _NS
    zbase = sid * rows
    nfull = rows // _C
    for t in range(nfull):
        pltpu.sync_copy(zz.at[pl.ds(0, _C)], acc.at[pl.ds(zbase + t * _C, _C)])
    rem = rows - nfull * _C
    if rem:
        pltpu.sync_copy(zz.at[pl.ds(0, rem)],
                        acc.at[pl.ds(zbase + nfull * _C, rem)])
    plsc.subcore_barrier()

    pltpu.sync_copy(idxd3.at[wid], ivd)
    sems = (sem0, sem1)

    def issue(j, b):
        pltpu.async_copy(payload.at[pl.ds(base + j * _C, _C)], buf.at[b],
                         sems[b])

    def drain(b):
        pltpu.make_async_copy(payload.at[pl.ds(0, _C)], buf.at[b],
                              sems[b]).wait()

    issue(0, 0)

    def step(i2, _):
        for b in (0, 1):
            j = i2 * 2 + b

            @pl.when(j + 1 < K)
            def _():
                issue(j + 1, 1 - b)

            drain(b)
            pltpu.sync_copy(buf.at[b], acc.at[ivd.at[j]], add=True)
        return _

    lax.fori_loop(0, K // 2, step, None)
    plsc.subcore_barrier()
    pltpu.sync_copy(acc.at[pl.ds(zbase, rows)],
                    acc_out.at[cid, pl.ds(zbase, rows)])


def _sc_scatter(payload, idxd3, zz, K, Np, tiled):
    D = payload.shape[1]
    kfn = pl.kernel(
        functools.partial(_scatter_body, K, Np, D),
        out_type=jax.ShapeDtypeStruct((_NC, Np, D), jnp.float32),
        mesh=_mesh(),
        compiler_params=_SC_TILED if tiled else _SC_LINEAR,
        scratch_types=[
            pltpu.VMEM((K, _C), jnp.int32),
            pltpu.VMEM((2, _C, D), jnp.float32),
            pltpu.VMEM_SHARED((Np, D), jnp.float32),
            pltpu.SemaphoreType.DMA, pltpu.SemaphoreType.DMA,
        ],
    )
    return kfn(payload, idxd3, zz)


def _edge_body(gA, gB, gx, wr, b1, W2T, b2, cW1T, cb1, cW2r,
               msgh_out, small_out):
    xdiff = gx[:, 0:3]
    radial = jnp.sum(xdiff * xdiff, axis=1, keepdims=True)
    u = xdiff / (jnp.sqrt(radial) + 1e-30)
    pre = gA[:] + gB[:] + radial * wr[:] + b1[:]
    m = _silu(pre)
    mh = _silu(jnp.dot(m, W2T[:], preferred_element_type=jnp.float32) + b2[:])
    c = _silu(jnp.dot(mh, cW1T[:], preferred_element_type=jnp.float32) + cb1[:])
    t = jnp.tanh(jnp.sum(c * cW2r[:], axis=1, keepdims=True))
    msgh_out[:] = mh
    mx = t * u * _COORDS_RANGE
    ones = jnp.ones_like(t)
    small_out[:] = jnp.concatenate(
        [mx, ones, jnp.zeros((t.shape[0], 4), t.dtype)], axis=1)


def _edge_mlp(gA, gB, gx, p):
    Ep, H = gA.shape
    wr = p['eW1'][:, -1].reshape(1, H)
    b1 = p['eb1'].reshape(1, H)
    blk = 2560
    full = lambda shape: pl.BlockSpec(shape, lambda i: (0, 0))
    return pl.pallas_call(
        _edge_body,
        grid=(Ep // blk,),
        in_specs=[
            pl.BlockSpec((blk, H), lambda i: (i, 0)),
            pl.BlockSpec((blk, H), lambda i: (i, 0)),
            pl.BlockSpec((blk, 16), lambda i: (i, 0)),
            full((1, H)), full((1, H)), full((H, H)), full((1, H)),
            full((H, H)), full((1, H)), full((1, H)),
        ],
        out_specs=[
            pl.BlockSpec((blk, H), lambda i: (i, 0)),
            pl.BlockSpec((blk, 8), lambda i: (i, 0)),
        ],
        out_shape=[
            jax.ShapeDtypeStruct((Ep, H), jnp.float32),
            jax.ShapeDtypeStruct((Ep, 8), jnp.float32),
        ],
    )(gA, gB, gx, wr, b1, p['eW2'].T, p['eb2'].reshape(1, H),
      p['cW1'].T, p['cb1'].reshape(1, H), p['cW2'].reshape(1, H))


def _proj_body(h, WaT, WbT, eA_out, eB_out):
    eA_out[:] = jnp.dot(h[:], WaT[:], preferred_element_type=jnp.float32)
    eB_out[:] = jnp.dot(h[:], WbT[:], preferred_element_type=jnp.float32)


def _edge_proj(h, eW1):
    N, F = h.shape
    H = eW1.shape[0]
    WaT = eW1[:, :F].T
    WbT = eW1[:, F:2 * F].T
    Fp = max(8, -(-F // 8) * 8)
    if Fp != F:
        h = jnp.pad(h, ((0, 0), (0, Fp - F)))
        WaT = jnp.pad(WaT, ((0, Fp - F), (0, 0)))
        WbT = jnp.pad(WbT, ((0, Fp - F), (0, 0)))
    blk = 2000 if N % 2000 == 0 else N
    return pl.pallas_call(
        _proj_body,
        grid=(N // blk,),
        in_specs=[
            pl.BlockSpec((blk, Fp), lambda i: (i, 0)),
            pl.BlockSpec((Fp, H), lambda i: (0, 0)),
            pl.BlockSpec((Fp, H), lambda i: (0, 0)),
        ],
        out_specs=[
            pl.BlockSpec((blk, H), lambda i: (i, 0)),
            pl.BlockSpec((blk, H), lambda i: (i, 0)),
        ],
        out_shape=[
            jax.ShapeDtypeStruct((N, H), jnp.float32),
            jax.ShapeDtypeStruct((N, H), jnp.float32),
        ],
    )(h, WaT, WbT)


def _node_body(h, hacc, hacc2, W1aT, W1bT, b1, W2T, b2, hout):
    hn = hacc[0] + hacc[1] + hacc2[0] + hacc2[1]
    z = (jnp.dot(h[:], W1aT[:], preferred_element_type=jnp.float32)
         + jnp.dot(hn, W1bT[:], preferred_element_type=jnp.float32) + b1[:])
    z = _silu(z)
    hout[:] = jnp.dot(z, W2T[:], preferred_element_type=jnp.float32) + b2[:]


def _node_update(h, hacc, hacc2, p):
    N, F = h.shape
    H = hacc.shape[2]
    OUTF = p['nW2'].shape[0]
    W1aT = p['nW1'][:, :F].T
    W1bT = p['nW1'][:, F:].T
    b1 = p['nb1'].reshape(1, -1)
    W2T = p['nW2'].T
    b2 = p['nb2'].reshape(1, -1)
    Fp = max(8, -(-F // 8) * 8)
    if Fp != F:
        h = jnp.pad(h, ((0, 0), (0, Fp - F)))
        W1aT = jnp.pad(W1aT, ((0, Fp - F), (0, 0)))
    blk = 2000 if N % 2000 == 0 else N
    return pl.pallas_call(
        _node_body,
        grid=(N // blk,),
        in_specs=[
            pl.BlockSpec((blk, Fp), lambda i: (i, 0)),
            pl.BlockSpec((2, blk, H), lambda i: (0, i, 0)),
            pl.BlockSpec((2, blk, H), lambda i: (0, i, 0)),
            pl.BlockSpec((Fp, H), lambda i: (0, 0)),
            pl.BlockSpec((H, H), lambda i: (0, 0)),
            pl.BlockSpec((1, H), lambda i: (0, 0)),
            pl.BlockSpec((H, OUTF), lambda i: (0, 0)),
            pl.BlockSpec((1, OUTF), lambda i: (0, 0)),
        ],
        out_specs=pl.BlockSpec((blk, OUTF), lambda i: (i, 0)),
        out_shape=jax.ShapeDtypeStruct((N, OUTF), jnp.float32),
    )(h, hacc, hacc2, W1aT, W1bT, b1, W2T, b2)


def kernel(node_feat, coord_feat, edge_index, params):
    src = edge_index[0].astype(jnp.int32)
    dst = edge_index[1].astype(jnp.int32)
    N = node_feat.shape[0]
    E = src.shape[0]

    K = -(-E // (_NW * _C))
    K = -(-K // 10) * 10
    Ep = _NW * _C * K
    K2 = 2 * K
    Np = -(-(N + 1) // _C) * _C

    pad = Ep - E
    srcp = jnp.pad(src, (0, pad))
    dstp = jnp.pad(dst, (0, pad))
    dstsp = jnp.pad(dst, (0, pad), constant_values=N)
    Eh = Ep // 2
    K2h, Kh = K2 // 2, K // 2
    halves = []
    for hh in (0, 1):
        sl = slice(hh * Eh, (hh + 1) * Eh)
        halves.append(dict(
            srcg2=srcp[sl].reshape(_NS, K2h, _C),
            dstg2=dstp[sl].reshape(_NS, K2h, _C),
            srcg3=srcp[sl].reshape(_NW, Kh, _C),
            dstg3=dstp[sl].reshape(_NW, Kh, _C),
            dsts3=dstsp[sl].reshape(_NW, Kh, _C),
        ))
    zh = jnp.zeros((_C, 128), jnp.float32)
    zs = jnp.zeros((_C, 8), jnp.float32)

    h, x = node_feat, coord_feat
    for p in params:
        eA, eB = _edge_proj(h, p['eW1'])
        XP = jnp.pad(x, ((0, 0), (0, 13)))
        g = [None, None]
        for hh in (0, 1):
            hv = halves[hh]
            gA, gB = _sc_gather(eA, eB, hv['srcg2'], hv['dstg2'], K2h, Eh)
            gx = _sc_gatherx(XP, hv['srcg3'], hv['dstg3'], Kh, Eh)
            g[hh] = (gA, gB, gx)
        ms = [_edge_mlp(*g[hh], p) for hh in (0, 1)]
        haccs, saccs = [], []
        for hh in (0, 1):
            msgh, small = ms[hh]
            haccs.append(_sc_scatter(msgh, halves[hh]['dsts3'], zh, Kh, Np,
                                     tiled=True))
            saccs.append(_sc_scatter(small, halves[hh]['dsts3'], zs, Kh, Np,
                                     tiled=False))
        sm = (saccs[0][0, :N] + saccs[0][1, :N]
              + saccs[1][0, :N] + saccs[1][1, :N])
        deg = jnp.maximum(sm[:, 3:4], 1.0)
        x = x + sm[:, 0:3] / deg
        h = _node_update(h, haccs[0], haccs[1], p)
    return (h, x)
